# Initial kernel scaffold; baseline (speedup 1.0000x reference)
import jax
import jax.numpy as jnp
from jax.experimental import pallas as pl


def kernel(x, edge_index, W1_nb, W1_self, b1, W2_nb, W2_self, b2, Wd, bd, Ws_nb, Ws_self, bs):
    raise NotImplementedError("write your pallas kernel here")


# speedup vs baseline: 4.3992x; 4.3992x over previous
import functools

import jax
import jax.numpy as jnp
from jax import lax
from jax.experimental import pallas as pl
from jax.experimental.pallas import tpu as pltpu
from jax.experimental.pallas import tpu_sc as plsc

N = 10000
E = 320000
D_IN = 128
D_H = 64

NC = 2
NS = 16
NW = NC * NS
BB = 128
KB = (E + NW * BB - 1) // (NW * BB)
KB = KB + (KB % 2)
E_PAD = NW * KB * BB
ROWS_PER_TILE = 632
N_ACC = NS * ROWS_PER_TILE
DEG_W = 16


def _sc_segsum_body(with_deg, *refs):
    if with_deg:
        (val_hbm, srcb_hbm, dstb_hbm, z64_hbm, z16_hbm, ones_hbm,
         out_hbm, deg_hbm,
         idx_s, idx_d, buf0, buf1, sem0, sem1, acc, ones_v, dacc) = refs
    else:
        (val_hbm, srcb_hbm, dstb_hbm, z64_hbm,
         out_hbm,
         idx_s, idx_d, buf0, buf1, sem0, sem1, acc) = refs

    c = lax.axis_index("c")
    s = lax.axis_index("s")
    wid = s * NC + c
    r0 = s * ROWS_PER_TILE

    pltpu.sync_copy(z64_hbm.at[pl.ds(r0, ROWS_PER_TILE)],
                    acc.at[pl.ds(r0, ROWS_PER_TILE)])
    if with_deg:
        pltpu.sync_copy(z16_hbm.at[pl.ds(r0, ROWS_PER_TILE)],
                        dacc.at[pl.ds(r0, ROWS_PER_TILE)])
        pltpu.sync_copy(ones_hbm, ones_v)

    rb0 = wid * KB
    pltpu.sync_copy(srcb_hbm.at[pl.ds(rb0, KB)], idx_s)
    pltpu.sync_copy(dstb_hbm.at[pl.ds(rb0, KB)], idx_d)

    plsc.subcore_barrier()

    bufs = (buf0, buf1)
    sems = (sem0, sem1)
    pltpu.async_copy(val_hbm.at[idx_s.at[0]], buf0, sem0)
    pltpu.async_copy(val_hbm.at[idx_s.at[1]], buf1, sem1)

    def step(i, carry):
        j2 = i * 2
        for b in range(2):
            j = j2 + b
            pltpu.make_async_copy(val_hbm.at[idx_s.at[j]], bufs[b], sems[b]).wait()
            nxt = j + 2

            @pl.when(nxt < KB)
            def _():
                pltpu.async_copy(val_hbm.at[idx_s.at[nxt]], bufs[b], sems[b])

            pltpu.sync_copy(bufs[b], acc.at[idx_d.at[j]], add=True)
            if with_deg:
                pltpu.sync_copy(ones_v, dacc.at[idx_d.at[j]], add=True)
        return carry

    lax.fori_loop(0, KB // 2, step, 0)

    plsc.subcore_barrier()

    pltpu.sync_copy(acc.at[pl.ds(r0, ROWS_PER_TILE)],
                    out_hbm.at[c, pl.ds(r0, ROWS_PER_TILE)])
    if with_deg:
        pltpu.sync_copy(dacc.at[pl.ds(r0, ROWS_PER_TILE)],
                        deg_hbm.at[c, pl.ds(r0, ROWS_PER_TILE)])


def _make_sc_pass(with_deg):
    out_type = [jax.ShapeDtypeStruct((NC, N_ACC, D_H), jnp.float32)]
    scratch = [
        pltpu.VMEM((KB, BB), jnp.int32),
        pltpu.VMEM((KB, BB), jnp.int32),
        pltpu.VMEM((BB, D_H), jnp.float32),
        pltpu.VMEM((BB, D_H), jnp.float32),
        pltpu.SemaphoreType.DMA,
        pltpu.SemaphoreType.DMA,
        pltpu.VMEM_SHARED((N_ACC, D_H), jnp.float32),
    ]
    if with_deg:
        out_type.append(jax.ShapeDtypeStruct((NC, N_ACC, DEG_W), jnp.float32))
        scratch += [
            pltpu.VMEM((BB, DEG_W), jnp.float32),
            pltpu.VMEM_SHARED((N_ACC, DEG_W), jnp.float32),
        ]
    mesh = plsc.VectorSubcoreMesh(core_axis_name="c", subcore_axis_name="s",
                                  num_cores=NC, num_subcores=NS)
    return pl.kernel(
        functools.partial(_sc_segsum_body, with_deg),
        out_type=tuple(out_type) if with_deg else out_type[0],
        mesh=mesh,
        scratch_types=tuple(scratch),
        compiler_params=pltpu.CompilerParams(use_tc_tiling_on_sc=False),
    )


BM = 2000


def _enc1_body(x_ref, wnb_ref, wself_ref, b_ref, p_ref, s_ref):
    xb = x_ref[...]
    p_ref[...] = jnp.dot(xb, wnb_ref[...], preferred_element_type=jnp.float32)
    s_ref[...] = (jnp.dot(xb, wself_ref[...], preferred_element_type=jnp.float32)
                  + b_ref[...])


def _mid_body(aggp_ref, degp_ref, s1_ref, wnb_ref, wself_ref, b_ref,
              p2_ref, s2_ref, d_ref):
    a = aggp_ref[0] + aggp_ref[1]
    d = (degp_ref[0] + degp_ref[1])[:, 0:1]
    h = jnp.maximum(a / jnp.maximum(d, 1.0) + s1_ref[...], 0.0)
    p2_ref[...] = jnp.dot(h, wnb_ref[...], preferred_element_type=jnp.float32)
    s2_ref[...] = (jnp.dot(h, wself_ref[...], preferred_element_type=jnp.float32)
                   + b_ref[...])
    d_ref[...] = d


def _dec_body(aggp_ref, d_ref, s2_ref, wsnb_ref, wsself_ref, bs_ref,
              wd_ref, bd_ref, p3_ref, s3_ref, xr_ref):
    a = aggp_ref[0] + aggp_ref[1]
    h = jnp.maximum(a / jnp.maximum(d_ref[...], 1.0) + s2_ref[...], 0.0)
    p3_ref[...] = jnp.dot(h, wsnb_ref[...], preferred_element_type=jnp.float32)
    s3_ref[...] = (jnp.dot(h, wsself_ref[...], preferred_element_type=jnp.float32)
                   + bs_ref[...])
    xr_ref[...] = (jnp.dot(h, wd_ref[...], preferred_element_type=jnp.float32)
                   + bd_ref[...])


def _hs_body(aggp_ref, d_ref, s3_ref, hs_ref):
    a = aggp_ref[0] + aggp_ref[1]
    hs_ref[...] = jnp.maximum(a / jnp.maximum(d_ref[...], 1.0) + s3_ref[...], 0.0)


BG = 1024


def _gram_body(a_ref, b_ref, out_ref):
    out_ref[...] = lax.dot_general(
        a_ref[...], b_ref[...], (((1,), (1,)), ((), ())),
        preferred_element_type=jnp.float32)


def _row_spec(width):
    return pl.BlockSpec((BM, width), lambda i: (i, 0))


def _agg_spec(width):
    return pl.BlockSpec((NC, BM, width), lambda i: (0, i, 0))


def _full_spec(shape):
    nd = len(shape)
    return pl.BlockSpec(shape, lambda i: (0,) * nd)


_GRID = (N // BM,)


def kernel(x, edge_index, W1_nb, W1_self, b1, W2_nb, W2_self, b2,
           Wd, bd, Ws_nb, Ws_self, bs):
    src = edge_index[0].astype(jnp.int32)
    dst = edge_index[1].astype(jnp.int32)
    pad = E_PAD - E
    srcb = jnp.concatenate([src, jnp.zeros((pad,), jnp.int32)]).reshape(-1, BB)
    dstb = jnp.concatenate([dst, jnp.full((pad,), N, jnp.int32)]).reshape(-1, BB)
    z64 = jnp.zeros((N_ACC, D_H), jnp.float32)
    z16 = jnp.zeros((N_ACC, DEG_W), jnp.float32)
    ones_col = jnp.zeros((BB, DEG_W), jnp.float32).at[:, 0].set(1.0)

    sc_pass1 = _make_sc_pass(True)
    sc_pass = _make_sc_pass(False)

    p1, s1 = pl.pallas_call(
        _enc1_body,
        grid=_GRID,
        in_specs=[_row_spec(D_IN), _full_spec((D_IN, D_H)),
                  _full_spec((D_IN, D_H)), _full_spec((D_H,))],
        out_specs=[_row_spec(D_H), _row_spec(D_H)],
        out_shape=[jax.ShapeDtypeStruct((N, D_H), jnp.float32)] * 2,
    )(x, W1_nb, W1_self, b1)

    agg1, degp = sc_pass1(p1, srcb, dstb, z64, z16, ones_col)

    p2, s2, dcol = pl.pallas_call(
        _mid_body,
        grid=_GRID,
        in_specs=[_agg_spec(D_H), _agg_spec(DEG_W), _row_spec(D_H),
                  _full_spec((D_H, D_H)), _full_spec((D_H, D_H)),
                  _full_spec((D_H,))],
        out_specs=[_row_spec(D_H), _row_spec(D_H), _row_spec(1)],
        out_shape=[jax.ShapeDtypeStruct((N, D_H), jnp.float32),
                   jax.ShapeDtypeStruct((N, D_H), jnp.float32),
                   jax.ShapeDtypeStruct((N, 1), jnp.float32)],
    )(agg1, degp, s1, W2_nb, W2_self, b2)

    agg2 = sc_pass(p2, srcb, dstb, z64)

    p3, s3, x_ = pl.pallas_call(
        _dec_body,
        grid=_GRID,
        in_specs=[_agg_spec(D_H), _row_spec(1), _row_spec(D_H),
                  _full_spec((D_H, D_H)), _full_spec((D_H, D_H)),
                  _full_spec((D_H,)), _full_spec((D_H, D_IN)),
                  _full_spec((D_IN,))],
        out_specs=[_row_spec(D_H), _row_spec(D_H), _row_spec(D_IN)],
        out_shape=[jax.ShapeDtypeStruct((N, D_H), jnp.float32),
                   jax.ShapeDtypeStruct((N, D_H), jnp.float32),
                   jax.ShapeDtypeStruct((N, D_IN), jnp.float32)],
    )(agg2, dcol, s2, Ws_nb, Ws_self, bs, Wd, bd)

    agg3 = sc_pass(p3, srcb, dstb, z64)

    hs = pl.pallas_call(
        _hs_body,
        grid=_GRID,
        in_specs=[_agg_spec(D_H), _row_spec(1), _row_spec(D_H)],
        out_specs=_row_spec(D_H),
        out_shape=jax.ShapeDtypeStruct((N, D_H), jnp.float32),
    )(agg3, dcol, s3)

    ng = (N + BG - 1) // BG
    s_ = pl.pallas_call(
        _gram_body,
        grid=(ng, ng),
        in_specs=[pl.BlockSpec((BG, D_H), lambda i, j: (i, 0)),
                  pl.BlockSpec((BG, D_H), lambda i, j: (j, 0))],
        out_specs=pl.BlockSpec((BG, BG), lambda i, j: (i, j)),
        out_shape=jax.ShapeDtypeStruct((N, N), jnp.float32),
    )(hs, hs)

    return (x_, s_)
